# baseline (speedup 1.0000x reference)
import functools

import jax
import jax.numpy as jnp
from jax import lax
from jax.experimental import pallas as pl
from jax.experimental.pallas import tpu as pltpu
from jax.experimental.pallas import tpu_sc as plsc

D = 64
LANES = 16
QROWS = 128

_DNUMS = lax.GatherDimensionNumbers(
    offset_dims=(), collapsed_slice_dims=(0,), start_index_map=(0,))


def _take16(v, perm):
    return lax.gather(v, perm[:, None], _DNUMS, slice_sizes=(1,),
                      mode=lax.GatherScatterMode.PROMISE_IN_BOUNDS)


def _allsum(v, lanes):
    for k in range(4):
        v = v + _take16(v, lanes ^ (1 << k))
    return v


def _rsqrt(s):
    bi = lax.bitcast_convert_type(s, jnp.int32)
    bi = jnp.int32(0x5F3759DF) - (bi >> 1)
    y = lax.bitcast_convert_type(bi, jnp.float32)
    half = jnp.float32(0.5) * s
    for _ in range(3):
        y = y * (jnp.float32(1.5) - half * y * y)
    return y


def _make_sc_call(B):
    info = plsc.get_sparse_core_info()
    NC, NS = info.num_cores, info.num_subcores
    NW = NC * NS
    b_per_w = B // NW
    n_quarters = b_per_w // QROWS
    groups_per_q = QROWS // LANES
    mesh = plsc.VectorSubcoreMesh(core_axis_name="c", subcore_axis_name="s")

    @functools.partial(
        pl.kernel,
        out_type=jax.ShapeDtypeStruct((B,), jnp.float32),
        mesh=mesh,
        scratch_types=[
            pltpu.VMEM((b_per_w,), jnp.int32),
            pltpu.VMEM((b_per_w,), jnp.int32),
            pltpu.VMEM((b_per_w,), jnp.int32),
            pltpu.VMEM((n_quarters, QROWS), jnp.int32),
            pltpu.VMEM((n_quarters, QROWS), jnp.int32),
            pltpu.VMEM((n_quarters, QROWS), jnp.int32),
            pltpu.VMEM((QROWS, 2 * D), jnp.float32),
            pltpu.VMEM((QROWS, 2 * D), jnp.float32),
            pltpu.VMEM((QROWS, 2 * D), jnp.float32),
            pltpu.VMEM((b_per_w,), jnp.float32),
            pltpu.SemaphoreType.DMA,
        ],
    )
    def sc_call(h_hbm, r_hbm, t_hbm, ent_hbm, rel_hbm, out_hbm,
                idx_h, idx_r, idx_t, row_h, row_r, row_t,
                h_buf, r_buf, t_buf, out_scr, sem):
        wid = lax.axis_index("s") * NC + lax.axis_index("c")
        base = wid * b_per_w
        lanes = lax.iota(jnp.int32, LANES)

        c1 = pltpu.async_copy(h_hbm.at[pl.ds(base, b_per_w)], idx_h, sem)
        c2 = pltpu.async_copy(r_hbm.at[pl.ds(base, b_per_w)], idx_r, sem)
        c3 = pltpu.async_copy(t_hbm.at[pl.ds(base, b_per_w)], idx_t, sem)
        c1.wait()
        c2.wait()
        c3.wait()

        for q in range(n_quarters):
            for k in range(QROWS // LANES):
                sl = pl.ds(q * QROWS + k * LANES, LANES)
                dsl = pl.ds(k * LANES, LANES)
                row_h[q, dsl] = idx_h[sl] >> 1
                row_r[q, dsl] = idx_r[sl] >> 1
                row_t[q, dsl] = idx_t[sl] >> 1

        def quarter_body(q, carry):
            g1 = pltpu.async_copy(ent_hbm.at[row_h.at[q]], h_buf, sem)
            g2 = pltpu.async_copy(rel_hbm.at[row_r.at[q]], r_buf, sem)
            g3 = pltpu.async_copy(ent_hbm.at[row_t.at[q]], t_buf, sem)
            g1.wait()
            g2.wait()
            g3.wait()

            def group_body(g, c):
                he = idx_h[pl.ds(q * QROWS + g * LANES, LANES)]
                re = idx_r[pl.ds(q * QROWS + g * LANES, LANES)]
                te = idx_t[pl.ds(q * QROWS + g * LANES, LANES)]
                acc = jnp.zeros((LANES,), jnp.float32)
                for j in range(LANES):
                    i = g * LANES + j
                    hoff = (he[j] & 1) * D
                    roff = (re[j] & 1) * D
                    toff = (te[j] & 1) * D
                    hv = [h_buf[i, pl.ds(hoff + kk * LANES, LANES)]
                          for kk in range(D // LANES)]
                    rv = [r_buf[i, pl.ds(roff + kk * LANES, LANES)]
                          for kk in range(D // LANES)]
                    tv = [t_buf[i, pl.ds(toff + kk * LANES, LANES)]
                          for kk in range(D // LANES)]
                    sh = _allsum(sum(v * v for v in hv), lanes)
                    sr = _allsum(sum(v * v for v in rv), lanes)
                    st = _allsum(sum(v * v for v in tv), lanes)
                    ih, ir, it = _rsqrt(sh), _rsqrt(sr), _rsqrt(st)
                    parts = [jnp.abs(a * ih + b * ir - d * it)
                             for a, b, d in zip(hv, rv, tv)]
                    sc = _allsum(parts[0] + parts[1] + parts[2] + parts[3],
                                 lanes)
                    acc = jnp.where(lanes == j, sc, acc)
                out_scr[pl.ds(q * QROWS + g * LANES, LANES)] = acc
                return c

            lax.fori_loop(0, groups_per_q, group_body, 0)
            return carry

        lax.fori_loop(0, n_quarters, quarter_body, 0)
        pltpu.sync_copy(out_scr, out_hbm.at[pl.ds(base, b_per_w)])

    return sc_call


def kernel(h, r, t, ent_table, rel_table):
    B = h.shape[0]
    V, d = ent_table.shape
    VR = rel_table.shape[0]
    ent2 = ent_table.reshape(V // 2, 2 * d)
    rel2 = rel_table.reshape(VR // 2, 2 * d)
    sc_call = _make_sc_call(B)
    score = sc_call(h.astype(jnp.int32), r.astype(jnp.int32),
                    t.astype(jnp.int32), ent2, rel2)
    return score[:, None]


# speedup vs baseline: 1.1166x; 1.1166x over previous
import functools

import jax
import jax.numpy as jnp
from jax import lax
from jax.experimental import pallas as pl
from jax.experimental.pallas import tpu as pltpu
from jax.experimental.pallas import tpu_sc as plsc

D = 64
LANES = 16
QROWS = 128

_DNUMS = lax.GatherDimensionNumbers(
    offset_dims=(), collapsed_slice_dims=(0,), start_index_map=(0,))


def _take16(v, perm):
    return lax.gather(v, perm[:, None], _DNUMS, slice_sizes=(1,),
                      mode=lax.GatherScatterMode.PROMISE_IN_BOUNDS)


def _allsum(v, lanes):
    for k in range(4):
        v = v + _take16(v, lanes ^ (1 << k))
    return v


def _rsqrt(s):
    bi = lax.bitcast_convert_type(s, jnp.int32)
    bi = jnp.int32(0x5F3759DF) - (bi >> 1)
    y = lax.bitcast_convert_type(bi, jnp.float32)
    half = jnp.float32(0.5) * s
    for _ in range(3):
        y = y * (jnp.float32(1.5) - half * y * y)
    return y


def _make_sc_call(B):
    info = plsc.get_sparse_core_info()
    NC, NS = info.num_cores, info.num_subcores
    NW = NC * NS
    b_per_w = B // NW
    n_quarters = b_per_w // QROWS
    groups_per_q = QROWS // LANES
    mesh = plsc.VectorSubcoreMesh(core_axis_name="c", subcore_axis_name="s")

    @functools.partial(
        pl.kernel,
        out_type=jax.ShapeDtypeStruct((B,), jnp.float32),
        mesh=mesh,
        scratch_types=[
            pltpu.VMEM((n_quarters, QROWS), jnp.int32),
            pltpu.VMEM((n_quarters, QROWS), jnp.int32),
            pltpu.VMEM((n_quarters, QROWS), jnp.int32),
            pltpu.VMEM((QROWS, 2 * D), jnp.float32),
            pltpu.VMEM((QROWS, 2 * D), jnp.float32),
            pltpu.VMEM((QROWS, 2 * D), jnp.float32),
            pltpu.VMEM((b_per_w,), jnp.float32),
            pltpu.SemaphoreType.DMA,
        ],
    )
    def sc_call(h_hbm, r_hbm, t_hbm, ent_hbm, rel_hbm, out_hbm,
                row_h, row_r, row_t, h_buf, r_buf, t_buf, out_scr, sem):
        wid = lax.axis_index("s") * NC + lax.axis_index("c")
        base = wid * b_per_w
        lanes = lax.iota(jnp.int32, LANES)

        cs = []
        for q in range(n_quarters):
            off = base + q * QROWS
            cs.append(pltpu.async_copy(
                h_hbm.at[pl.ds(off, QROWS)], row_h.at[q], sem))
            cs.append(pltpu.async_copy(
                r_hbm.at[pl.ds(off, QROWS)], row_r.at[q], sem))
            cs.append(pltpu.async_copy(
                t_hbm.at[pl.ds(off, QROWS)], row_t.at[q], sem))
        for c in cs:
            c.wait()

        def quarter_body(q, carry):
            g1 = pltpu.async_copy(ent_hbm.at[row_h.at[q]], h_buf, sem)
            g2 = pltpu.async_copy(rel_hbm.at[row_r.at[q]], r_buf, sem)
            g3 = pltpu.async_copy(ent_hbm.at[row_t.at[q]], t_buf, sem)
            g1.wait()
            g2.wait()
            g3.wait()

            def group_body(g, c):
                acc = jnp.zeros((LANES,), jnp.float32)
                for j in range(LANES):
                    i = g * LANES + j
                    hv = [h_buf[i, pl.ds(kk * LANES, LANES)]
                          for kk in range(D // LANES)]
                    rv = [r_buf[i, pl.ds(kk * LANES, LANES)]
                          for kk in range(D // LANES)]
                    tv = [t_buf[i, pl.ds(kk * LANES, LANES)]
                          for kk in range(D // LANES)]
                    sh = _allsum(sum(v * v for v in hv), lanes)
                    sr = _allsum(sum(v * v for v in rv), lanes)
                    st = _allsum(sum(v * v for v in tv), lanes)
                    ih, ir, it = _rsqrt(sh), _rsqrt(sr), _rsqrt(st)
                    parts = [jnp.abs(a * ih + b * ir - d * it)
                             for a, b, d in zip(hv, rv, tv)]
                    sc = _allsum(parts[0] + parts[1] + parts[2] + parts[3],
                                 lanes)
                    acc = jnp.where(lanes == j, sc, acc)
                out_scr[pl.ds(q * QROWS + g * LANES, LANES)] = acc
                return c

            lax.fori_loop(0, groups_per_q, group_body, 0)
            return carry

        lax.fori_loop(0, n_quarters, quarter_body, 0)
        pltpu.sync_copy(out_scr, out_hbm.at[pl.ds(base, b_per_w)])

    return sc_call


def kernel(h, r, t, ent_table, rel_table):
    B = h.shape[0]
    ent_pad = jnp.pad(ent_table, ((0, 0), (0, D)))
    rel_pad = jnp.pad(rel_table, ((0, 0), (0, D)))
    sc_call = _make_sc_call(B)
    score = sc_call(h.astype(jnp.int32), r.astype(jnp.int32),
                    t.astype(jnp.int32), ent_pad, rel_pad)
    return score[:, None]
